# initial kernel scaffold (unmeasured)
import jax
import jax.numpy as jnp
from jax import lax
from jax.experimental import pallas as pl
from jax.experimental.pallas import tpu as pltpu

N_DEV = 8
E_LOCAL = 8
N_TOK = 2048
D_IN = 512
D_OUT = 1024
N_EXP = 64
CHUNK = N_TOK // N_DEV


def kernel(x, router_W, route_idx, expert_W, shared_W):
    def body(x_ref, router_ref, idx_ref, ew_ref, sw_ref, out_ref,
             comm_ref, rs_send, rs_recv, ag_send, ag_recv):
        my = lax.axis_index("i")
        left = lax.rem(my - 1 + N_DEV, N_DEV)
        right = lax.rem(my + 1, N_DEV)

        barrier_sem = pltpu.get_barrier_semaphore()
        for nbr in (left, right):
            pl.semaphore_signal(
                barrier_sem, inc=1,
                device_id=(nbr,), device_id_type=pl.DeviceIdType.MESH,
            )
        pl.semaphore_wait(barrier_sem, 2)

        xv = x_ref[...]
        scores = jnp.dot(xv, router_ref[...], preferred_element_type=jnp.float32)
        m = jnp.max(scores, axis=-1, keepdims=True)
        e = jnp.exp(scores - m)
        probs = e / jnp.sum(e, axis=-1, keepdims=True)

        idx = idx_ref[...]
        exp_iota = lax.broadcasted_iota(jnp.int32, (1, N_EXP), 1)
        onehot = (idx == exp_iota).astype(jnp.float32)
        p_sel = jnp.sum(probs * onehot, axis=1, keepdims=True)

        for j in range(E_LOCAL):
            e_id = my * E_LOCAL + j
            coef = jnp.where(idx == e_id, p_sel, 0.0)
            contrib = jnp.dot(coef * xv, ew_ref[j],
                              preferred_element_type=jnp.float32)
            if j == 0:
                out_ref[...] = contrib
            else:
                out_ref[...] += contrib

        for s in range(N_DEV - 1):
            c_send = lax.rem(my - s + N_DEV, N_DEV)
            rdma = pltpu.make_async_remote_copy(
                src_ref=out_ref.at[pl.ds(c_send * CHUNK, CHUNK), :],
                dst_ref=comm_ref.at[s],
                send_sem=rs_send.at[s],
                recv_sem=rs_recv.at[s],
                device_id=(right,),
                device_id_type=pl.DeviceIdType.MESH,
            )
            rdma.start()
            rdma.wait()
            c_recv = lax.rem(my - s - 1 + N_DEV, N_DEV)
            off = c_recv * CHUNK
            out_ref[pl.ds(off, CHUNK), :] = (
                out_ref[pl.ds(off, CHUNK), :] + comm_ref[s]
            )

        c_own = lax.rem(my + 1, N_DEV)
        own_off = c_own * CHUNK
        out_ref[pl.ds(own_off, CHUNK), :] += jnp.dot(
            xv[pl.ds(own_off, CHUNK), :], sw_ref[...],
            preferred_element_type=jnp.float32,
        )

        for s in range(N_DEV - 1):
            c_ag = lax.rem(my + 1 - s + N_DEV, N_DEV)
            off = c_ag * CHUNK
            rdma = pltpu.make_async_remote_copy(
                src_ref=out_ref.at[pl.ds(off, CHUNK), :],
                dst_ref=out_ref.at[pl.ds(off, CHUNK), :],
                send_sem=ag_send.at[s],
                recv_sem=ag_recv.at[s],
                device_id=(right,),
                device_id_type=pl.DeviceIdType.MESH,
            )
            rdma.start()
            rdma.wait()

    return pl.pallas_call(
        body,
        out_shape=jax.ShapeDtypeStruct((N_TOK, D_OUT), jnp.float32),
        in_specs=[
            pl.BlockSpec(memory_space=pltpu.VMEM),
            pl.BlockSpec(memory_space=pltpu.VMEM),
            pl.BlockSpec(memory_space=pltpu.VMEM),
            pl.BlockSpec(memory_space=pltpu.VMEM),
            pl.BlockSpec(memory_space=pltpu.VMEM),
        ],
        out_specs=pl.BlockSpec(memory_space=pltpu.VMEM),
        scratch_shapes=[
            pltpu.VMEM((N_DEV - 1, CHUNK, D_OUT), jnp.float32),
            pltpu.SemaphoreType.DMA((N_DEV - 1,)),
            pltpu.SemaphoreType.DMA((N_DEV - 1,)),
            pltpu.SemaphoreType.DMA((N_DEV - 1,)),
            pltpu.SemaphoreType.DMA((N_DEV - 1,)),
        ],
        compiler_params=pltpu.CompilerParams(collective_id=0),
    )(x, router_W, route_idx, expert_W, shared_W)


# baseline (device time: 229622 ns/iter reference)
import jax
import jax.numpy as jnp
from jax import lax
from jax.experimental import pallas as pl
from jax.experimental.pallas import tpu as pltpu

N_DEV = 8
E_LOCAL = 8
N_TOK = 2048
D_IN = 512
D_OUT = 1024
N_EXP = 64
CHUNK = N_TOK // N_DEV


def kernel(x, router_W, route_idx, expert_W, shared_W):
    def body(x_ref, router_ref, idx_ref, ew_ref, sw_ref, out_ref,
             comm_ref, rs_send, rs_recv, ag_send, ag_recv):
        my = lax.axis_index("i")
        left = lax.rem(my - 1 + N_DEV, N_DEV)
        right = lax.rem(my + 1, N_DEV)

        barrier_sem = pltpu.get_barrier_semaphore()
        for nbr in (left, right):
            pl.semaphore_signal(
                barrier_sem, inc=1,
                device_id=(nbr,), device_id_type=pl.DeviceIdType.MESH,
            )
        pl.semaphore_wait(barrier_sem, 2)

        xv = x_ref[...]
        scores = jnp.dot(xv, router_ref[...], preferred_element_type=jnp.float32)
        m = jnp.max(scores, axis=-1, keepdims=True)
        e = jnp.exp(scores - m)
        probs = e / jnp.sum(e, axis=-1, keepdims=True)

        idx = idx_ref[...]
        exp_iota = lax.broadcasted_iota(jnp.int32, (1, N_EXP), 1)
        onehot = (idx == exp_iota).astype(jnp.float32)
        p_sel = jnp.sum(probs * onehot, axis=1, keepdims=True)

        for j in range(E_LOCAL):
            e_id = my * E_LOCAL + j
            coef = jnp.where(idx == e_id, p_sel, 0.0)
            contrib = jnp.dot(coef * xv, ew_ref[j],
                              preferred_element_type=jnp.float32)
            if j == 0:
                out_ref[...] = contrib
            else:
                out_ref[...] += contrib

        for s in range(N_DEV - 1):
            c_send = lax.rem(my - s + N_DEV, N_DEV)
            rdma = pltpu.make_async_remote_copy(
                src_ref=out_ref.at[pl.ds(c_send * CHUNK, CHUNK), :],
                dst_ref=comm_ref.at[s],
                send_sem=rs_send.at[s],
                recv_sem=rs_recv.at[s],
                device_id=(right,),
                device_id_type=pl.DeviceIdType.MESH,
            )
            rdma.start()
            rdma.wait()
            c_recv = lax.rem(my - s - 1 + N_DEV, N_DEV)
            off = c_recv * CHUNK
            out_ref[pl.ds(off, CHUNK), :] = (
                out_ref[pl.ds(off, CHUNK), :] + comm_ref[s]
            )

        c_own = lax.rem(my + 1, N_DEV)
        own_off = c_own * CHUNK
        out_ref[pl.ds(own_off, CHUNK), :] += jnp.dot(
            x_ref[pl.ds(own_off, CHUNK), :], sw_ref[...],
            preferred_element_type=jnp.float32,
        )

        for s in range(N_DEV - 1):
            c_ag = lax.rem(my + 1 - s + N_DEV, N_DEV)
            off = c_ag * CHUNK
            rdma = pltpu.make_async_remote_copy(
                src_ref=out_ref.at[pl.ds(off, CHUNK), :],
                dst_ref=out_ref.at[pl.ds(off, CHUNK), :],
                send_sem=ag_send.at[s],
                recv_sem=ag_recv.at[s],
                device_id=(right,),
                device_id_type=pl.DeviceIdType.MESH,
            )
            rdma.start()
            rdma.wait()

    return pl.pallas_call(
        body,
        out_shape=jax.ShapeDtypeStruct((N_TOK, D_OUT), jnp.float32),
        in_specs=[
            pl.BlockSpec(memory_space=pltpu.VMEM),
            pl.BlockSpec(memory_space=pltpu.VMEM),
            pl.BlockSpec(memory_space=pltpu.VMEM),
            pl.BlockSpec(memory_space=pltpu.VMEM),
            pl.BlockSpec(memory_space=pltpu.VMEM),
        ],
        out_specs=pl.BlockSpec(memory_space=pltpu.VMEM),
        scratch_shapes=[
            pltpu.VMEM((N_DEV - 1, CHUNK, D_OUT), jnp.float32),
            pltpu.SemaphoreType.DMA((N_DEV - 1,)),
            pltpu.SemaphoreType.DMA((N_DEV - 1,)),
            pltpu.SemaphoreType.DMA((N_DEV - 1,)),
            pltpu.SemaphoreType.DMA((N_DEV - 1,)),
        ],
        compiler_params=pltpu.CompilerParams(
            collective_id=0,
            vmem_limit_bytes=100 * 1024 * 1024,
        ),
    )(x, router_W, route_idx, expert_W, shared_W)


# device time: 117825 ns/iter; 1.9488x vs baseline; 1.9488x over previous
import jax
import jax.numpy as jnp
from jax import lax
from jax.experimental import pallas as pl
from jax.experimental.pallas import tpu as pltpu

N_DEV = 8
E_LOCAL = 8
N_TOK = 2048
D_IN = 512
D_OUT = 1024
N_EXP = 64

PARTS = ((0, 384, (0, 1, 2)), (384, 384, (1, 2, 0)), (768, 256, (2, 0, 1)))
RS_HALF = (1024, 512, 256)
STAGE_OFF = (0, 1024, 1536)
N_STEPS = 3


def kernel(x, router_W, route_idx, expert_W, shared_W):
    def body(x_ref, router_ref, idx_ref, ew_ref, sw_ref, out_ref,
             comm_ref, rs_send, rs_recv, ag_send, ag_recv):
        my = lax.axis_index("i")

        m4 = lax.rem(my, 4)
        cx = jnp.where((m4 == 1) | (m4 == 2), 1, 0).astype(jnp.int32)
        cy = jnp.where(m4 >= 2, 1, 0).astype(jnp.int32)
        cz = lax.div(my, 4)

        def pos(px, py, pz):
            return pz * 4 + 2 * py + jnp.bitwise_xor(px, py)

        coord = (cx, cy, cz)
        nbrs = (pos(1 - cx, cy, cz), pos(cx, 1 - cy, cz), pos(cx, cy, 1 - cz))

        xv = x_ref[...]
        scores = jnp.dot(xv, router_ref[...], preferred_element_type=jnp.float32)
        m = jnp.max(scores, axis=-1, keepdims=True)
        e = jnp.exp(scores - m)
        probs = e / jnp.sum(e, axis=-1, keepdims=True)

        idx = idx_ref[...]
        exp_iota = lax.broadcasted_iota(jnp.int32, (1, N_EXP), 1)
        onehot = (idx == exp_iota).astype(jnp.float32)
        p_sel = jnp.sum(probs * onehot, axis=1, keepdims=True)

        for j in range(E_LOCAL):
            e_id = my * E_LOCAL + j
            coef = jnp.where(idx == e_id, p_sel, 0.0)
            contrib = jnp.dot(coef * xv, ew_ref[j],
                              preferred_element_type=jnp.float32)
            if j == 0:
                out_ref[...] = contrib
            else:
                out_ref[...] += contrib

        barrier_sem = pltpu.get_barrier_semaphore()
        for nbr in nbrs:
            pl.semaphore_signal(
                barrier_sem, inc=1,
                device_id=(nbr,), device_id_type=pl.DeviceIdType.MESH,
            )
        pl.semaphore_wait(barrier_sem, 3)

        starts = [jnp.int32(0), jnp.int32(0), jnp.int32(0)]
        for k in range(N_STEPS):
            half = RS_HALF[k]
            rdmas = []
            for p, (o, w, order) in enumerate(PARTS):
                ax = order[k]
                c = coord[ax]
                keep = starts[p] + c * half
                send = starts[p] + (1 - c) * half
                rdma = pltpu.make_async_remote_copy(
                    src_ref=out_ref.at[pl.ds(send, half), pl.ds(o, w)],
                    dst_ref=comm_ref.at[pl.ds(STAGE_OFF[k], half), pl.ds(o, w)],
                    send_sem=rs_send.at[k, p],
                    recv_sem=rs_recv.at[k, p],
                    device_id=(nbrs[ax],),
                    device_id_type=pl.DeviceIdType.MESH,
                )
                rdma.start()
                rdmas.append(rdma)
                starts[p] = keep
            for rdma in rdmas:
                rdma.wait()
            for p, (o, w, order) in enumerate(PARTS):
                out_ref[pl.ds(starts[p], half), pl.ds(o, w)] += (
                    comm_ref[pl.ds(STAGE_OFF[k], half), pl.ds(o, w)]
                )

        for p, (o, w, order) in enumerate(PARTS):
            out_ref[pl.ds(starts[p], 256), pl.ds(o, w)] += jnp.dot(
                x_ref[pl.ds(starts[p], 256), :],
                sw_ref[:, pl.ds(o, w)],
                preferred_element_type=jnp.float32,
            )

        for k in range(N_STEPS):
            cur_len = 256 << k
            rdmas = []
            for p, (o, w, order) in enumerate(PARTS):
                ax = order[2 - k]
                c = coord[ax]
                rdma = pltpu.make_async_remote_copy(
                    src_ref=out_ref.at[pl.ds(starts[p], cur_len), pl.ds(o, w)],
                    dst_ref=out_ref.at[pl.ds(starts[p], cur_len), pl.ds(o, w)],
                    send_sem=ag_send.at[k, p],
                    recv_sem=ag_recv.at[k, p],
                    device_id=(nbrs[ax],),
                    device_id_type=pl.DeviceIdType.MESH,
                )
                rdma.start()
                rdmas.append(rdma)
                starts[p] = starts[p] - c * cur_len
            for rdma in rdmas:
                rdma.wait()

    return pl.pallas_call(
        body,
        out_shape=jax.ShapeDtypeStruct((N_TOK, D_OUT), jnp.float32),
        in_specs=[
            pl.BlockSpec(memory_space=pltpu.VMEM),
            pl.BlockSpec(memory_space=pltpu.VMEM),
            pl.BlockSpec(memory_space=pltpu.VMEM),
            pl.BlockSpec(memory_space=pltpu.VMEM),
            pl.BlockSpec(memory_space=pltpu.VMEM),
        ],
        out_specs=pl.BlockSpec(memory_space=pltpu.VMEM),
        scratch_shapes=[
            pltpu.VMEM((1792, D_OUT), jnp.float32),
            pltpu.SemaphoreType.DMA((N_STEPS, 3)),
            pltpu.SemaphoreType.DMA((N_STEPS, 3)),
            pltpu.SemaphoreType.DMA((N_STEPS, 3)),
            pltpu.SemaphoreType.DMA((N_STEPS, 3)),
        ],
        compiler_params=pltpu.CompilerParams(
            collective_id=0,
            vmem_limit_bytes=100 * 1024 * 1024,
        ),
    )(x, router_W, route_idx, expert_W, shared_W)


# device time: 115439 ns/iter; 1.9891x vs baseline; 1.0207x over previous
import jax
import jax.numpy as jnp
from jax import lax
from jax.experimental import pallas as pl
from jax.experimental.pallas import tpu as pltpu

N_DEV = 8
E_LOCAL = 8
N_TOK = 2048
D_IN = 512
D_OUT = 1024
N_EXP = 64

PARTS = ((0, 384, (0, 1, 2)), (384, 384, (1, 2, 0)), (768, 256, (2, 0, 1)))
RS_HALF = (1024, 512, 256)
STAGE_OFF = (0, 1024, 1536)
N_STEPS = 3


def kernel(x, router_W, route_idx, expert_W, shared_W):
    def body(x_ref, router_ref, idx_ref, ew_ref, sw_ref, out_ref,
             comm_ref, rs_send, rs_recv, ag_send, ag_recv):
        my = lax.axis_index("i")

        m4 = lax.rem(my, 4)
        cx = jnp.where((m4 == 1) | (m4 == 2), 1, 0).astype(jnp.int32)
        cy = jnp.where(m4 >= 2, 1, 0).astype(jnp.int32)
        cz = lax.div(my, 4)

        def pos(px, py, pz):
            return pz * 4 + 2 * py + jnp.bitwise_xor(px, py)

        coord = (cx, cy, cz)
        nbrs = (pos(1 - cx, cy, cz), pos(cx, 1 - cy, cz), pos(cx, cy, 1 - cz))

        barrier_sem = pltpu.get_barrier_semaphore()
        for nbr in nbrs:
            pl.semaphore_signal(
                barrier_sem, inc=1,
                device_id=(nbr,), device_id_type=pl.DeviceIdType.MESH,
            )
        pl.semaphore_wait(barrier_sem, 3)

        xv = x_ref[...]
        scores = jnp.dot(xv, router_ref[...], preferred_element_type=jnp.float32)
        m = jnp.max(scores, axis=-1, keepdims=True)
        e = jnp.exp(scores - m)
        probs = e / jnp.sum(e, axis=-1, keepdims=True)

        idx = idx_ref[...]
        exp_iota = lax.broadcasted_iota(jnp.int32, (1, N_EXP), 1)
        onehot = (idx == exp_iota).astype(jnp.float32)
        p_sel = jnp.sum(probs * onehot, axis=1, keepdims=True)

        starts = [jnp.int32(0)] * 3
        rdmas = {}

        def start_rs(k, p):
            o, w, order = PARTS[p]
            half = RS_HALF[k]
            c = coord[order[k]]
            send = starts[p] + (1 - c) * half
            rdma = pltpu.make_async_remote_copy(
                src_ref=out_ref.at[pl.ds(send, half), pl.ds(o, w)],
                dst_ref=comm_ref.at[pl.ds(STAGE_OFF[k], half), pl.ds(o, w)],
                send_sem=rs_send.at[k, p],
                recv_sem=rs_recv.at[k, p],
                device_id=(nbrs[order[k]],),
                device_id_type=pl.DeviceIdType.MESH,
            )
            rdma.start()
            rdmas[("rs", k, p)] = rdma
            starts[p] = starts[p] + c * half

        def finish_rs(k, p):
            o, w, _ = PARTS[p]
            half = RS_HALF[k]
            rdmas[("rs", k, p)].wait()
            out_ref[pl.ds(starts[p], half), pl.ds(o, w)] += (
                comm_ref[pl.ds(STAGE_OFF[k], half), pl.ds(o, w)]
            )

        def start_ag(k, p):
            o, w, order = PARTS[p]
            cur_len = 256 << k
            c = coord[order[2 - k]]
            rdma = pltpu.make_async_remote_copy(
                src_ref=out_ref.at[pl.ds(starts[p], cur_len), pl.ds(o, w)],
                dst_ref=out_ref.at[pl.ds(starts[p], cur_len), pl.ds(o, w)],
                send_sem=ag_send.at[k, p],
                recv_sem=ag_recv.at[k, p],
                device_id=(nbrs[order[2 - k]],),
                device_id_type=pl.DeviceIdType.MESH,
            )
            rdma.start()
            rdmas[("ag", k, p)] = rdma
            starts[p] = starts[p] - c * cur_len

        for p, (o, w, order) in enumerate(PARTS):
            for j in range(E_LOCAL):
                coef = jnp.where(idx == my * E_LOCAL + j, p_sel, 0.0)
                contrib = coef * jnp.dot(xv, ew_ref[j, :, pl.ds(o, w)],
                                         preferred_element_type=jnp.float32)
                if j == 0:
                    out_ref[:, pl.ds(o, w)] = contrib
                else:
                    out_ref[:, pl.ds(o, w)] += contrib
            start_rs(0, p)

        for k in range(1, N_STEPS):
            for p in range(3):
                finish_rs(k - 1, p)
                start_rs(k, p)

        for p, (o, w, _) in enumerate(PARTS):
            finish_rs(N_STEPS - 1, p)
            out_ref[pl.ds(starts[p], 256), pl.ds(o, w)] += jnp.dot(
                x_ref[pl.ds(starts[p], 256), :],
                sw_ref[:, pl.ds(o, w)],
                preferred_element_type=jnp.float32,
            )
            start_ag(0, p)

        for k in range(1, N_STEPS):
            for p in range(3):
                rdmas[("ag", k - 1, p)].wait()
                start_ag(k, p)
        for p in range(3):
            rdmas[("ag", N_STEPS - 1, p)].wait()

    return pl.pallas_call(
        body,
        out_shape=jax.ShapeDtypeStruct((N_TOK, D_OUT), jnp.float32),
        in_specs=[
            pl.BlockSpec(memory_space=pltpu.VMEM),
            pl.BlockSpec(memory_space=pltpu.VMEM),
            pl.BlockSpec(memory_space=pltpu.VMEM),
            pl.BlockSpec(memory_space=pltpu.VMEM),
            pl.BlockSpec(memory_space=pltpu.VMEM),
        ],
        out_specs=pl.BlockSpec(memory_space=pltpu.VMEM),
        scratch_shapes=[
            pltpu.VMEM((1792, D_OUT), jnp.float32),
            pltpu.SemaphoreType.DMA((N_STEPS, 3)),
            pltpu.SemaphoreType.DMA((N_STEPS, 3)),
            pltpu.SemaphoreType.DMA((N_STEPS, 3)),
            pltpu.SemaphoreType.DMA((N_STEPS, 3)),
        ],
        compiler_params=pltpu.CompilerParams(
            collective_id=0,
            vmem_limit_bytes=100 * 1024 * 1024,
        ),
    )(x, router_W, route_idx, expert_W, shared_W)


# device time: 82150 ns/iter; 2.7952x vs baseline; 1.4052x over previous
import jax
import jax.numpy as jnp
from jax import lax
from jax.experimental import pallas as pl
from jax.experimental.pallas import tpu as pltpu

N_DEV = 8
E_LOCAL = 8
N_TOK = 2048
D_IN = 512
D_OUT = 1024
N_EXP = 64

PARTS = ((0, 768, (0, 1, 2)), (768, 768, (1, 2, 0)), (1536, 512, (2, 0, 1)))
STAGE_OFF = ((0, 384, 576), (672, 1056, 1248), (1344, 1600, 1728))
N_STEPS = 3


def kernel(x, router_W, route_idx, expert_W, shared_W):
    def body(x_ref, router_ref, idx_ref, ew_ref, sw_ref, out_ref,
             comm_ref, wire_ref, rs_send, rs_recv, ag_send, ag_recv):
        my = lax.axis_index("i")

        m4 = lax.rem(my, 4)
        cx = jnp.where((m4 == 1) | (m4 == 2), 1, 0).astype(jnp.int32)
        cy = jnp.where(m4 >= 2, 1, 0).astype(jnp.int32)
        cz = lax.div(my, 4)

        def pos(px, py, pz):
            return pz * 4 + 2 * py + jnp.bitwise_xor(px, py)

        coord = (cx, cy, cz)
        nbrs = (pos(1 - cx, cy, cz), pos(cx, 1 - cy, cz), pos(cx, cy, 1 - cz))

        barrier_sem = pltpu.get_barrier_semaphore()
        for nbr in nbrs:
            pl.semaphore_signal(
                barrier_sem, inc=1,
                device_id=(nbr,), device_id_type=pl.DeviceIdType.MESH,
            )
        pl.semaphore_wait(barrier_sem, 3)

        xv = x_ref[...]
        scores = jnp.dot(xv, router_ref[...], preferred_element_type=jnp.float32)
        m = jnp.max(scores, axis=-1, keepdims=True)
        e = jnp.exp(scores - m)
        probs = e / jnp.sum(e, axis=-1, keepdims=True)

        idx = idx_ref[...]
        exp_iota = lax.broadcasted_iota(jnp.int32, (1, N_EXP), 1)
        onehot = (idx == exp_iota).astype(jnp.float32)
        p_sel = jnp.sum(probs * onehot, axis=1, keepdims=True)

        starts = [jnp.int32(PARTS[p][0]) for p in range(3)]
        rdmas = {}

        def start_rs(k, p):
            o, L, order = PARTS[p]
            half = L >> (k + 1)
            c = coord[order[k]]
            send = starts[p] + (1 - c) * half
            wire_ref[pl.ds(STAGE_OFF[p][k], half), :] = (
                out_ref[pl.ds(send, half), :].astype(jnp.bfloat16)
            )
            rdma = pltpu.make_async_remote_copy(
                src_ref=wire_ref.at[pl.ds(STAGE_OFF[p][k], half), :],
                dst_ref=comm_ref.at[pl.ds(STAGE_OFF[p][k], half), :],
                send_sem=rs_send.at[k, p],
                recv_sem=rs_recv.at[k, p],
                device_id=(nbrs[order[k]],),
                device_id_type=pl.DeviceIdType.MESH,
            )
            rdma.start()
            rdmas[("rs", k, p)] = rdma
            starts[p] = starts[p] + c * half

        def finish_rs(k, p):
            _, L, _ = PARTS[p]
            half = L >> (k + 1)
            rdmas[("rs", k, p)].wait()
            out_ref[pl.ds(starts[p], half), :] += (
                comm_ref[pl.ds(STAGE_OFF[p][k], half), :].astype(jnp.float32)
            )

        def start_ag(k, p):
            _, L, order = PARTS[p]
            cur_len = L >> (N_STEPS - k)
            c = coord[order[2 - k]]
            rdma = pltpu.make_async_remote_copy(
                src_ref=wire_ref.at[pl.ds(starts[p], cur_len), :],
                dst_ref=wire_ref.at[pl.ds(starts[p], cur_len), :],
                send_sem=ag_send.at[k, p],
                recv_sem=ag_recv.at[k, p],
                device_id=(nbrs[order[2 - k]],),
                device_id_type=pl.DeviceIdType.MESH,
            )
            rdma.start()
            rdmas[("ag", k, p)] = rdma
            starts[p] = starts[p] - c * cur_len
            rdmas[("agband", k, p)] = (starts[p] + (1 - c) * cur_len, cur_len)

        def finish_ag(k, p):
            rdmas[("ag", k, p)].wait()
            ps, cur_len = rdmas[("agband", k, p)]
            out_ref[pl.ds(ps, cur_len), :] = (
                wire_ref[pl.ds(ps, cur_len), :].astype(jnp.float32)
            )

        for p, (o, L, order) in enumerate(PARTS):
            xp = x_ref[pl.ds(o, L), :]
            idx_p = idx_ref[pl.ds(o, L), :]
            psel_p = p_sel[o:o + L, :]
            for j in range(E_LOCAL):
                coef = jnp.where(idx_p == my * E_LOCAL + j, psel_p, 0.0)
                contrib = coef * jnp.dot(xp, ew_ref[j],
                                         preferred_element_type=jnp.float32)
                if j == 0:
                    out_ref[pl.ds(o, L), :] = contrib
                else:
                    out_ref[pl.ds(o, L), :] += contrib
            start_rs(0, p)

        for k in range(1, N_STEPS):
            for p in range(3):
                finish_rs(k - 1, p)
                start_rs(k, p)

        for p, (_, L, _) in enumerate(PARTS):
            finish_rs(N_STEPS - 1, p)
            own = L >> N_STEPS
            out_ref[pl.ds(starts[p], own), :] += jnp.dot(
                x_ref[pl.ds(starts[p], own), :], sw_ref[...],
                preferred_element_type=jnp.float32,
            )
            wire_ref[pl.ds(starts[p], own), :] = (
                out_ref[pl.ds(starts[p], own), :].astype(jnp.bfloat16)
            )
            start_ag(0, p)

        for k in range(1, N_STEPS):
            for p in range(3):
                rdmas[("ag", k - 1, p)].wait()
                start_ag(k, p)
                ps, cl = rdmas[("agband", k - 1, p)]
                out_ref[pl.ds(ps, cl), :] = (
                    wire_ref[pl.ds(ps, cl), :].astype(jnp.float32)
                )
        for p in range(3):
            finish_ag(N_STEPS - 1, p)

    return pl.pallas_call(
        body,
        out_shape=jax.ShapeDtypeStruct((N_TOK, D_OUT), jnp.float32),
        in_specs=[
            pl.BlockSpec(memory_space=pltpu.VMEM),
            pl.BlockSpec(memory_space=pltpu.VMEM),
            pl.BlockSpec(memory_space=pltpu.VMEM),
            pl.BlockSpec(memory_space=pltpu.VMEM),
            pl.BlockSpec(memory_space=pltpu.VMEM),
        ],
        out_specs=pl.BlockSpec(memory_space=pltpu.VMEM),
        scratch_shapes=[
            pltpu.VMEM((1792, D_OUT), jnp.bfloat16),
            pltpu.VMEM((N_TOK, D_OUT), jnp.bfloat16),
            pltpu.SemaphoreType.DMA((N_STEPS, 3)),
            pltpu.SemaphoreType.DMA((N_STEPS, 3)),
            pltpu.SemaphoreType.DMA((N_STEPS, 3)),
            pltpu.SemaphoreType.DMA((N_STEPS, 3)),
        ],
        compiler_params=pltpu.CompilerParams(
            collective_id=0,
            vmem_limit_bytes=100 * 1024 * 1024,
        ),
    )(x, router_W, route_idx, expert_W, shared_W)
